# baseline (speedup 1.0000x reference)
import functools

import jax
import jax.numpy as jnp
import numpy as np
from jax import lax
from jax.experimental import pallas as pl
from jax.experimental.pallas import tpu as pltpu
from jax.experimental.pallas import tpu_sc as plsc

N = 10000
E = 320000
H = 128
G = 64
NLAYERS = 4

_NC = 2
_NS = 16
_EPAD = 327680
_EPW = _EPAD // (_NC * _NS)
_CHUNK = 128
_NCHNK = _EPW // _CHUNK
_NPAD = 10240
_RPT = _NPAD // _NS

_BNS = float(1.0 / np.sqrt(np.float32(1.0 + 1e-5)))


def _sc_agg_kernel(h_hbm, src_hbm, dst_hbm, zero_hbm, out_hbm,
                   srcv, dstv, rows0, acc, sem0):
    c = lax.axis_index("c")
    s = lax.axis_index("s")
    pltpu.sync_copy(zero_hbm.at[pl.ds(s * _RPT, _RPT)],
                    acc.at[pl.ds(s * _RPT, _RPT)])
    plsc.subcore_barrier()

    pltpu.sync_copy(src_hbm.at[c, s], srcv)
    pltpu.sync_copy(dst_hbm.at[c, s], dstv)

    def body(j, carry):
        pltpu.async_copy(h_hbm.at[srcv.at[j]], rows0, sem0).wait()
        pltpu.sync_copy(rows0, acc.at[dstv.at[j]], add=True)
        return carry

    lax.fori_loop(0, _NCHNK, body, 0)
    plsc.subcore_barrier()
    pltpu.sync_copy(acc.at[pl.ds(s * _RPT, _RPT)],
                    out_hbm.at[c, pl.ds(s * _RPT, _RPT)])


def _sc_agg(h, src4, dst4, zeros):
    mesh = plsc.VectorSubcoreMesh(core_axis_name="c", subcore_axis_name="s")
    f = functools.partial(
        pl.kernel,
        out_type=jax.ShapeDtypeStruct((_NC, _NPAD, H), jnp.float32),
        mesh=mesh,
        scratch_types=[
            pltpu.VMEM((_NCHNK, _CHUNK), jnp.int32),
            pltpu.VMEM((_NCHNK, _CHUNK), jnp.int32),
            pltpu.VMEM((_CHUNK, H), jnp.float32),
            pltpu.VMEM_SHARED((_NPAD, H), jnp.float32),
            pltpu.SemaphoreType.DMA,
        ],
    )(_sc_agg_kernel)
    return f(h, src4, dst4, zeros)


_BR = 2000


def _embed_body(x_ref, w_ref, b_ref, o_ref):
    o_ref[...] = jnp.maximum(
        jnp.dot(x_ref[...], w_ref[...], preferred_element_type=jnp.float32)
        + b_ref[...], 0.0)


def _embed(x, w, b):
    return pl.pallas_call(
        _embed_body,
        grid=(N // _BR,),
        in_specs=[
            pl.BlockSpec((_BR, H), lambda i: (i, 0)),
            pl.BlockSpec((H, H), lambda i: (0, 0)),
            pl.BlockSpec((1, H), lambda i: (0, 0)),
        ],
        out_specs=pl.BlockSpec((_BR, H), lambda i: (i, 0)),
        out_shape=jax.ShapeDtypeStruct((N, H), jnp.float32),
    )(x, w, b)


def _layer_body(scale_ref, h_ref, a0_ref, a1_ref, w1_ref, b1_ref, g1_ref,
                t1_ref, w2_ref, b2_ref, ng_ref, nb_ref, o_ref):
    h = h_ref[...]
    z = scale_ref[0, 0] * h + a0_ref[...] + a1_ref[...]
    t = jnp.dot(z, w1_ref[...], preferred_element_type=jnp.float32) + b1_ref[...]
    t = jnp.maximum(t * (_BNS * g1_ref[...]) + t1_ref[...], 0.0)
    u = jnp.dot(t, w2_ref[...], preferred_element_type=jnp.float32) + b2_ref[...]
    u = jnp.maximum(u * (_BNS * ng_ref[...]) + nb_ref[...], 0.0)
    o_ref[...] = u + h


def _layer(scale, h, a0, a1, lp):
    return pl.pallas_call(
        _layer_body,
        grid=(N // _BR,),
        in_specs=[
            pl.BlockSpec((1, 1), lambda i: (0, 0)),
            pl.BlockSpec((_BR, H), lambda i: (i, 0)),
            pl.BlockSpec((_BR, H), lambda i: (i, 0)),
            pl.BlockSpec((_BR, H), lambda i: (i, 0)),
            pl.BlockSpec((H, 2 * H), lambda i: (0, 0)),
            pl.BlockSpec((1, 2 * H), lambda i: (0, 0)),
            pl.BlockSpec((1, 2 * H), lambda i: (0, 0)),
            pl.BlockSpec((1, 2 * H), lambda i: (0, 0)),
            pl.BlockSpec((2 * H, H), lambda i: (0, 0)),
            pl.BlockSpec((1, H), lambda i: (0, 0)),
            pl.BlockSpec((1, H), lambda i: (0, 0)),
            pl.BlockSpec((1, H), lambda i: (0, 0)),
        ],
        out_specs=pl.BlockSpec((_BR, H), lambda i: (i, 0)),
        out_shape=jax.ShapeDtypeStruct((N, H), jnp.float32),
    )(scale, h, a0, a1,
      lp['W1'], lp['b1'].reshape(1, -1), lp['bn1_g'].reshape(1, -1),
      lp['bn1_b'].reshape(1, -1), lp['W2'], lp['b2'].reshape(1, -1),
      lp['n_g'].reshape(1, -1), lp['n_b'].reshape(1, -1))


_PBR = 1000


def _pool_body(bsm_ref, oh_ref, bf_ref, h1_ref, h2_ref, h3_ref, h4_ref,
               sum_ref, cnt_ref, max_ref):
    i = pl.program_id(0)

    @pl.when(i == 0)
    def _():
        sum_ref[...] = jnp.zeros_like(sum_ref)
        cnt_ref[...] = jnp.zeros_like(cnt_ref)
        max_ref[...] = jnp.full_like(max_ref, -jnp.inf)

    hj = jnp.concatenate(
        [h1_ref[...], h2_ref[...], h3_ref[...], h4_ref[...]], axis=1)
    oh = oh_ref[...]
    dn = (((0,), (0,)), ((), ()))
    sum_ref[...] += lax.dot_general(oh, hj, dn,
                                    preferred_element_type=jnp.float32)
    ones = jnp.ones((oh.shape[0], 1), jnp.float32)
    cnt_ref[...] += jnp.broadcast_to(
        lax.dot_general(oh, ones, dn, preferred_element_type=jnp.float32),
        cnt_ref.shape)
    lo = bsm_ref[i * _PBR]
    hi = bsm_ref[i * _PBR + _PBR - 1]
    bf = bf_ref[...]

    def gbody(g, carry):
        sel = jnp.where(bf == g.astype(jnp.float32), hj, -jnp.inf)
        pmax = jnp.max(sel, axis=0, keepdims=True)
        max_ref[pl.ds(g, 1), :] = jnp.maximum(max_ref[pl.ds(g, 1), :], pmax)
        return carry

    lax.fori_loop(lo, hi + 1, gbody, 0)


def _pool(batch_col, onehot, batchf, hs):
    return pl.pallas_call(
        _pool_body,
        grid=(N // _PBR,),
        in_specs=[
            pl.BlockSpec(memory_space=pltpu.SMEM),
            pl.BlockSpec((_PBR, G), lambda i: (i, 0)),
            pl.BlockSpec((_PBR, 1), lambda i: (i, 0)),
            pl.BlockSpec((_PBR, H), lambda i: (i, 0)),
            pl.BlockSpec((_PBR, H), lambda i: (i, 0)),
            pl.BlockSpec((_PBR, H), lambda i: (i, 0)),
            pl.BlockSpec((_PBR, H), lambda i: (i, 0)),
        ],
        out_specs=[
            pl.BlockSpec((G, NLAYERS * H), lambda i: (0, 0)),
            pl.BlockSpec((G, H), lambda i: (0, 0)),
            pl.BlockSpec((G, NLAYERS * H), lambda i: (0, 0)),
        ],
        out_shape=[
            jax.ShapeDtypeStruct((G, NLAYERS * H), jnp.float32),
            jax.ShapeDtypeStruct((G, H), jnp.float32),
            jax.ShapeDtypeStruct((G, NLAYERS * H), jnp.float32),
        ],
    )(batch_col, onehot, batchf, *hs)


def _head_body(sum_ref, cnt_ref, max_ref, w1, b1, g1, t1, w2, b2, g2, t2,
               w3, b3, g3, t3, w4, b4, o_ref):
    cnt = cnt_ref[:, 0:1]
    mean = sum_ref[...] / jnp.maximum(cnt, 1.0)
    mx = max_ref[...]
    mx = jnp.where(jnp.isfinite(mx), mx, 0.0)
    z = jnp.concatenate([mean, mx], axis=1)
    z = jnp.dot(z, w1[...], preferred_element_type=jnp.float32) + b1[...]
    z = jnp.maximum(z * (_BNS * g1[...]) + t1[...], 0.0)
    z = jnp.dot(z, w2[...], preferred_element_type=jnp.float32) + b2[...]
    z = jnp.maximum(z * (_BNS * g2[...]) + t2[...], 0.0)
    z = jnp.dot(z, w3[...], preferred_element_type=jnp.float32) + b3[...]
    z = jnp.maximum(z * (_BNS * g3[...]) + t3[...], 0.0)
    o_ref[...] = jnp.dot(z, w4[...], preferred_element_type=jnp.float32) + b4[...]


def _head(sums, cnt, mx, p):
    args = [sums, cnt, mx]
    for i in (1, 2, 3):
        args += [p['W%d' % i], p['b%d' % i].reshape(1, -1),
                 p['g%d' % i].reshape(1, -1), p['bt%d' % i].reshape(1, -1)]
    args += [p['W4'], p['b4'].reshape(1, -1)]
    return pl.pallas_call(
        _head_body,
        out_shape=jax.ShapeDtypeStruct((G, 1), jnp.float32),
    )(*args)


def kernel(x, edge_index, batch, params):
    nw = _NC * _NS
    ppw = _EPW - E // nw
    pad_src = (jnp.arange(nw * ppw, dtype=edge_index.dtype) % N
               ).reshape(nw, ppw)
    pad_dst = (N + jnp.arange(nw * ppw, dtype=edge_index.dtype) % (_NPAD - N)
               ).reshape(nw, ppw)
    src4 = jnp.concatenate(
        [edge_index[0].reshape(nw, -1), pad_src], axis=1
    ).reshape(_NC, _NS, _NCHNK, _CHUNK)
    dst4 = jnp.concatenate(
        [edge_index[1].reshape(nw, -1), pad_dst], axis=1
    ).reshape(_NC, _NS, _NCHNK, _CHUNK)
    zeros = jnp.zeros((_NPAD, H), jnp.float32)

    h = _embed(x, params['W_emb'], params['b_emb'].reshape(1, -1))
    outs = []
    for lp in params['layers']:
        agg = _sc_agg(h, src4, dst4, zeros)
        scale = (1.0 + lp['eps']).astype(jnp.float32).reshape(1, 1)
        h = _layer(scale, h, agg[0], agg[1], lp)
        outs.append(h)

    batch_col = batch
    onehot = (batch[:, None] == jnp.arange(G, dtype=batch.dtype)[None, :]
              ).astype(jnp.float32)
    batchf = batch.astype(jnp.float32).reshape(N, 1)
    sums, cnt, mx = _pool(batch_col, onehot, batchf, outs)
    return _head(sums, cnt, mx, params['pred'])


# speedup vs baseline: 1.0018x; 1.0018x over previous
import functools

import jax
import jax.numpy as jnp
import numpy as np
from jax import lax
from jax.experimental import pallas as pl
from jax.experimental.pallas import tpu as pltpu
from jax.experimental.pallas import tpu_sc as plsc

N = 10000
E = 320000
H = 128
G = 64
NLAYERS = 4

_NC = 2
_NS = 16
_EPAD = 327680
_EPW = _EPAD // (_NC * _NS)
_CHUNK = 128
_NCHNK = _EPW // _CHUNK
_NPAD = 10240
_RPT = _NPAD // _NS

_BNS = float(1.0 / np.sqrt(np.float32(1.0 + 1e-5)))


def _sc_agg_kernel(h_hbm, src_hbm, dst_hbm, zero_hbm, out_hbm,
                   srcv, dstv, rows0, acc, sem0, sem1):
    c = lax.axis_index("c")
    s = lax.axis_index("s")
    pltpu.sync_copy(zero_hbm.at[pl.ds(s * _RPT, _RPT)],
                    acc.at[pl.ds(s * _RPT, _RPT)])
    plsc.subcore_barrier()

    pltpu.sync_copy(src_hbm.at[c, s], srcv)
    pltpu.sync_copy(dst_hbm.at[c, s], dstv)

    pltpu.async_copy(h_hbm.at[srcv.at[0]], rows0, sem0).wait()
    pltpu.async_copy(rows0, acc.at[dstv.at[0]], sem1, add=True)

    def body(j, carry):
        pltpu.make_async_copy(rows0, acc.at[dstv.at[j]], sem1).wait()
        pltpu.async_copy(h_hbm.at[srcv.at[j]], rows0, sem0).wait()
        pltpu.async_copy(rows0, acc.at[dstv.at[j]], sem1, add=True)
        return carry

    lax.fori_loop(1, _NCHNK, body, 0)
    pltpu.make_async_copy(rows0, acc.at[dstv.at[0]], sem1).wait()
    plsc.subcore_barrier()
    pltpu.sync_copy(acc.at[pl.ds(s * _RPT, _RPT)],
                    out_hbm.at[c, pl.ds(s * _RPT, _RPT)])


def _sc_agg(h, src4, dst4, zeros):
    mesh = plsc.VectorSubcoreMesh(core_axis_name="c", subcore_axis_name="s")
    f = functools.partial(
        pl.kernel,
        out_type=jax.ShapeDtypeStruct((_NC, _NPAD, H), jnp.float32),
        mesh=mesh,
        scratch_types=[
            pltpu.VMEM((_NCHNK, _CHUNK), jnp.int32),
            pltpu.VMEM((_NCHNK, _CHUNK), jnp.int32),
            pltpu.VMEM((_CHUNK, H), jnp.float32),
            pltpu.VMEM_SHARED((_NPAD, H), jnp.float32),
            pltpu.SemaphoreType.DMA,
            pltpu.SemaphoreType.DMA,
        ],
    )(_sc_agg_kernel)
    return f(h, src4, dst4, zeros)


_BR = 2000


def _embed_body(x_ref, w_ref, b_ref, o_ref):
    o_ref[...] = jnp.maximum(
        jnp.dot(x_ref[...], w_ref[...], preferred_element_type=jnp.float32)
        + b_ref[...], 0.0)


def _embed(x, w, b):
    return pl.pallas_call(
        _embed_body,
        grid=(N // _BR,),
        in_specs=[
            pl.BlockSpec((_BR, H), lambda i: (i, 0)),
            pl.BlockSpec((H, H), lambda i: (0, 0)),
            pl.BlockSpec((1, H), lambda i: (0, 0)),
        ],
        out_specs=pl.BlockSpec((_BR, H), lambda i: (i, 0)),
        out_shape=jax.ShapeDtypeStruct((N, H), jnp.float32),
    )(x, w, b)


def _layer_body(scale_ref, h_ref, a0_ref, a1_ref, w1_ref, b1_ref, g1_ref,
                t1_ref, w2_ref, b2_ref, ng_ref, nb_ref, o_ref):
    h = h_ref[...]
    z = scale_ref[0, 0] * h + a0_ref[...] + a1_ref[...]
    t = jnp.dot(z, w1_ref[...], preferred_element_type=jnp.float32) + b1_ref[...]
    t = jnp.maximum(t * (_BNS * g1_ref[...]) + t1_ref[...], 0.0)
    u = jnp.dot(t, w2_ref[...], preferred_element_type=jnp.float32) + b2_ref[...]
    u = jnp.maximum(u * (_BNS * ng_ref[...]) + nb_ref[...], 0.0)
    o_ref[...] = u + h


def _layer(scale, h, a0, a1, lp):
    return pl.pallas_call(
        _layer_body,
        grid=(N // _BR,),
        in_specs=[
            pl.BlockSpec((1, 1), lambda i: (0, 0)),
            pl.BlockSpec((_BR, H), lambda i: (i, 0)),
            pl.BlockSpec((_BR, H), lambda i: (i, 0)),
            pl.BlockSpec((_BR, H), lambda i: (i, 0)),
            pl.BlockSpec((H, 2 * H), lambda i: (0, 0)),
            pl.BlockSpec((1, 2 * H), lambda i: (0, 0)),
            pl.BlockSpec((1, 2 * H), lambda i: (0, 0)),
            pl.BlockSpec((1, 2 * H), lambda i: (0, 0)),
            pl.BlockSpec((2 * H, H), lambda i: (0, 0)),
            pl.BlockSpec((1, H), lambda i: (0, 0)),
            pl.BlockSpec((1, H), lambda i: (0, 0)),
            pl.BlockSpec((1, H), lambda i: (0, 0)),
        ],
        out_specs=pl.BlockSpec((_BR, H), lambda i: (i, 0)),
        out_shape=jax.ShapeDtypeStruct((N, H), jnp.float32),
    )(scale, h, a0, a1,
      lp['W1'], lp['b1'].reshape(1, -1), lp['bn1_g'].reshape(1, -1),
      lp['bn1_b'].reshape(1, -1), lp['W2'], lp['b2'].reshape(1, -1),
      lp['n_g'].reshape(1, -1), lp['n_b'].reshape(1, -1))


_PBR = 1000


def _pool_body(bsm_ref, oh_ref, bf_ref, h1_ref, h2_ref, h3_ref, h4_ref,
               sum_ref, cnt_ref, max_ref):
    i = pl.program_id(0)

    @pl.when(i == 0)
    def _():
        sum_ref[...] = jnp.zeros_like(sum_ref)
        cnt_ref[...] = jnp.zeros_like(cnt_ref)
        max_ref[...] = jnp.full_like(max_ref, -jnp.inf)

    hj = jnp.concatenate(
        [h1_ref[...], h2_ref[...], h3_ref[...], h4_ref[...]], axis=1)
    oh = oh_ref[...]
    dn = (((0,), (0,)), ((), ()))
    sum_ref[...] += lax.dot_general(oh, hj, dn,
                                    preferred_element_type=jnp.float32)
    ones = jnp.ones((oh.shape[0], 1), jnp.float32)
    cnt_ref[...] += jnp.broadcast_to(
        lax.dot_general(oh, ones, dn, preferred_element_type=jnp.float32),
        cnt_ref.shape)
    lo = bsm_ref[i * _PBR]
    hi = bsm_ref[i * _PBR + _PBR - 1]
    bf = bf_ref[...]

    def gbody(g, carry):
        sel = jnp.where(bf == g.astype(jnp.float32), hj, -jnp.inf)
        pmax = jnp.max(sel, axis=0, keepdims=True)
        max_ref[pl.ds(g, 1), :] = jnp.maximum(max_ref[pl.ds(g, 1), :], pmax)
        return carry

    lax.fori_loop(lo, hi + 1, gbody, 0)


def _pool(batch_col, onehot, batchf, hs):
    return pl.pallas_call(
        _pool_body,
        grid=(N // _PBR,),
        in_specs=[
            pl.BlockSpec(memory_space=pltpu.SMEM),
            pl.BlockSpec((_PBR, G), lambda i: (i, 0)),
            pl.BlockSpec((_PBR, 1), lambda i: (i, 0)),
            pl.BlockSpec((_PBR, H), lambda i: (i, 0)),
            pl.BlockSpec((_PBR, H), lambda i: (i, 0)),
            pl.BlockSpec((_PBR, H), lambda i: (i, 0)),
            pl.BlockSpec((_PBR, H), lambda i: (i, 0)),
        ],
        out_specs=[
            pl.BlockSpec((G, NLAYERS * H), lambda i: (0, 0)),
            pl.BlockSpec((G, H), lambda i: (0, 0)),
            pl.BlockSpec((G, NLAYERS * H), lambda i: (0, 0)),
        ],
        out_shape=[
            jax.ShapeDtypeStruct((G, NLAYERS * H), jnp.float32),
            jax.ShapeDtypeStruct((G, H), jnp.float32),
            jax.ShapeDtypeStruct((G, NLAYERS * H), jnp.float32),
        ],
    )(batch_col, onehot, batchf, *hs)


def _head_body(sum_ref, cnt_ref, max_ref, w1, b1, g1, t1, w2, b2, g2, t2,
               w3, b3, g3, t3, w4, b4, o_ref):
    cnt = cnt_ref[:, 0:1]
    mean = sum_ref[...] / jnp.maximum(cnt, 1.0)
    mx = max_ref[...]
    mx = jnp.where(jnp.isfinite(mx), mx, 0.0)
    z = jnp.concatenate([mean, mx], axis=1)
    z = jnp.dot(z, w1[...], preferred_element_type=jnp.float32) + b1[...]
    z = jnp.maximum(z * (_BNS * g1[...]) + t1[...], 0.0)
    z = jnp.dot(z, w2[...], preferred_element_type=jnp.float32) + b2[...]
    z = jnp.maximum(z * (_BNS * g2[...]) + t2[...], 0.0)
    z = jnp.dot(z, w3[...], preferred_element_type=jnp.float32) + b3[...]
    z = jnp.maximum(z * (_BNS * g3[...]) + t3[...], 0.0)
    o_ref[...] = jnp.dot(z, w4[...], preferred_element_type=jnp.float32) + b4[...]


def _head(sums, cnt, mx, p):
    args = [sums, cnt, mx]
    for i in (1, 2, 3):
        args += [p['W%d' % i], p['b%d' % i].reshape(1, -1),
                 p['g%d' % i].reshape(1, -1), p['bt%d' % i].reshape(1, -1)]
    args += [p['W4'], p['b4'].reshape(1, -1)]
    return pl.pallas_call(
        _head_body,
        out_shape=jax.ShapeDtypeStruct((G, 1), jnp.float32),
    )(*args)


def kernel(x, edge_index, batch, params):
    nw = _NC * _NS
    ppw = _EPW - E // nw
    pad_src = (jnp.arange(nw * ppw, dtype=edge_index.dtype) % N
               ).reshape(nw, ppw)
    pad_dst = (N + jnp.arange(nw * ppw, dtype=edge_index.dtype) % (_NPAD - N)
               ).reshape(nw, ppw)
    src4 = jnp.concatenate(
        [edge_index[0].reshape(nw, -1), pad_src], axis=1
    ).reshape(_NC, _NS, _NCHNK, _CHUNK)
    dst4 = jnp.concatenate(
        [edge_index[1].reshape(nw, -1), pad_dst], axis=1
    ).reshape(_NC, _NS, _NCHNK, _CHUNK)
    zeros = jnp.zeros((_NPAD, H), jnp.float32)

    h = _embed(x, params['W_emb'], params['b_emb'].reshape(1, -1))
    outs = []
    for lp in params['layers']:
        agg = _sc_agg(h, src4, dst4, zeros)
        scale = (1.0 + lp['eps']).astype(jnp.float32).reshape(1, 1)
        h = _layer(scale, h, agg[0], agg[1], lp)
        outs.append(h)

    batch_col = batch
    onehot = (batch[:, None] == jnp.arange(G, dtype=batch.dtype)[None, :]
              ).astype(jnp.float32)
    batchf = batch.astype(jnp.float32).reshape(N, 1)
    sums, cnt, mx = _pool(batch_col, onehot, batchf, outs)
    return _head(sums, cnt, mx, params['pred'])
